# baseline (speedup 1.0000x reference)
import functools

import numpy as np
import jax
import jax.numpy as jnp
from jax import lax
from jax.experimental import pallas as pl
from jax.experimental.pallas import tpu as pltpu
from jax.experimental.pallas import tpu_sc as plsc

N = 262144
NLEV = 16
T = 1 << 19
BASE_RES = 16
SCALE = 1.3819128799
P1 = np.int32(np.uint32(2654435761).astype(np.int32))
P2 = np.int32(805459861)
OUT_D = 3 + 2 * NLEV

_info = plsc.get_sparse_core_info()
NC = _info.num_cores
NW = _info.num_cores * _info.num_subcores
NPW = N // NW
C = 512
NCHUNK = NPW // C
NSL = C // 16
GB = 128
NB = 8 * C // GB

LEVELS = []
for _l in range(NLEV):
    _res = int(np.floor(BASE_RES * (SCALE ** _l)))
    LEVELS.append((_l, _res, (_res + 1) ** 3 <= T))


_mesh = plsc.VectorSubcoreMesh(core_axis_name="c", subcore_axis_name="s")


@functools.partial(
    pl.kernel,
    out_type=jax.ShapeDtypeStruct((N, OUT_D), jnp.float32),
    mesh=_mesh,
    scratch_types=[
        pltpu.VMEM((C, 3), jnp.float32),
        pltpu.VMEM((8 * C,), jnp.int32),
        pltpu.VMEM((8, C), jnp.float32),
        pltpu.VMEM((8 * C, 2), jnp.float32),
        pltpu.VMEM((C, OUT_D), jnp.float32),
        pltpu.SemaphoreType.DMA,
    ],
    compiler_params=pltpu.CompilerParams(
        needs_layout_passes=False, use_tc_tiling_on_sc=False
    ),
)
def _encode_sc(x_hbm, grid_hbm, out_hbm, x_v, idx_v, w_v, rows_v, out_v, sem):
    wid = lax.axis_index("s") * NC + lax.axis_index("c")
    iota = lax.iota(jnp.int32, 16)
    z16 = jnp.zeros((16,), jnp.int32)
    o16 = jnp.ones((16,), jnp.int32)

    def chunk_body(ci, carry):
        base = wid * NPW + ci * C
        pltpu.sync_copy(x_hbm.at[pl.ds(base, C)], x_v)

        def xcopy(s, c):
            rid = s * 16 + iota
            for d in range(3):
                cd = jnp.full((16,), d, jnp.int32)
                xd = plsc.load_gather(x_v, [rid, cd])
                plsc.store_scatter(out_v, [rid, cd], xd)
            return c

        lax.fori_loop(0, NSL, xcopy, 0)

        for (l, res, dense) in LEVELS:
            S = res + 1

            def pa(s, c, l=l, res=res, dense=dense, S=S):
                rid = s * 16 + iota
                xs = [
                    plsc.load_gather(x_v, [rid, jnp.full((16,), d, jnp.int32)])
                    for d in range(3)
                ]
                pos = [xd * jnp.float32(res) for xd in xs]
                p0 = [p.astype(jnp.int32) for p in pos]
                fr = [p - q.astype(jnp.float32) for p, q in zip(pos, p0)]
                if dense:
                    tx = [p0[0] + (l * T), p0[0] + (l * T + 1)]
                    ty = [p0[1] * S, (p0[1] + 1) * S]
                    tz = [p0[2] * (S * S), (p0[2] + 1) * (S * S)]
                else:
                    tx = [p0[0], p0[0] + 1]
                    ty = [p0[1] * P1, (p0[1] + 1) * P1]
                    tz = [p0[2] * P2, (p0[2] + 1) * P2]
                wx = [1.0 - fr[0], fr[0]]
                wy = [1.0 - fr[1], fr[1]]
                wz = [1.0 - fr[2], fr[2]]
                for corner in range(8):
                    i, j, k = corner & 1, (corner >> 1) & 1, (corner >> 2) & 1
                    if dense:
                        idx = tx[i] + ty[j] + tz[k]
                    else:
                        idx = ((tx[i] ^ ty[j] ^ tz[k]) & (T - 1)) + l * T
                    w = wx[i] * wy[j] * wz[k]
                    idx_v[pl.ds(corner * C + s * 16, 16)] = idx
                    w_v[corner, pl.ds(s * 16, 16)] = w
                return c

            lax.fori_loop(0, NSL, pa, 0)

            def issue(b, c):
                pltpu.async_copy(
                    grid_hbm.at[idx_v.at[pl.ds(b * GB, GB)]],
                    rows_v.at[pl.ds(b * GB, GB)],
                    sem,
                )
                return c

            lax.fori_loop(0, NB, issue, 0)

            def drain(b, c):
                pltpu.make_async_copy(
                    grid_hbm.at[idx_v.at[pl.ds(b * GB, GB)]],
                    rows_v.at[pl.ds(b * GB, GB)],
                    sem,
                ).wait()
                return c

            lax.fori_loop(0, NB, drain, 0)

            col0 = 3 + 2 * l

            def pc(s, c, col0=col0):
                rid = s * 16 + iota
                acc0 = jnp.zeros((16,), jnp.float32)
                acc1 = jnp.zeros((16,), jnp.float32)
                for corner in range(8):
                    rr = corner * C + s * 16 + iota
                    g0 = plsc.load_gather(rows_v, [rr, z16])
                    g1 = plsc.load_gather(rows_v, [rr, o16])
                    w = w_v[corner, pl.ds(s * 16, 16)]
                    acc0 = acc0 + w * g0
                    acc1 = acc1 + w * g1
                plsc.store_scatter(
                    out_v, [rid, jnp.full((16,), col0, jnp.int32)], acc0
                )
                plsc.store_scatter(
                    out_v, [rid, jnp.full((16,), col0 + 1, jnp.int32)], acc1
                )
                return c

            lax.fori_loop(0, NSL, pc, 0)

        pltpu.sync_copy(out_v, out_hbm.at[pl.ds(base, C)])
        return carry

    lax.fori_loop(0, NCHUNK, chunk_body, 0)


def kernel(x, grid):
    return _encode_sc(x, grid.reshape(NLEV * T, 2))
